# baseline (speedup 1.0000x reference)
import functools

import jax
import jax.numpy as jnp
from jax import lax
from jax.experimental import pallas as pl
from jax.experimental.pallas import tpu as pltpu
from jax.experimental.pallas import tpu_sc as plsc


def _make_emb_kernel(B, V, D, NC, NS):
    d_per_c = D // NC
    b_per_s = B // NS

    mesh = plsc.VectorSubcoreMesh(core_axis_name="c", subcore_axis_name="s")

    @functools.partial(
        pl.kernel,
        out_type=jax.ShapeDtypeStruct((4, 8, B), jnp.float32),
        mesh=mesh,
        scratch_types=[
            pltpu.VMEM((B,), jnp.int32),
            pltpu.VMEM((b_per_s,), jnp.float32),
            pltpu.VMEM_SHARED((V,), jnp.float32),
            pltpu.SemaphoreType.DMA,
            pltpu.SemaphoreType.DMA,
        ],
        compiler_params=pltpu.CompilerParams(
            needs_layout_passes=False, use_tc_tiling_on_sc=False
        ),
    )
    def emb_kernel(idx_hbm, tab2, out3, idx_v, vals_v, row_sp, sem, osem):
        cid = lax.axis_index("c")
        sid = lax.axis_index("s")
        pltpu.sync_copy(idx_hbm, idx_v)
        writes = []
        for i in range(d_per_c):
            d = cid * d_per_c + i

            @pl.when(sid == 0)
            def _():
                pltpu.sync_copy(tab2.at[d], row_sp)

            plsc.subcore_barrier()
            pltpu.async_copy(
                row_sp.at[idx_v.at[pl.ds(sid * b_per_s, b_per_s)]],
                vals_v,
                sem,
            ).wait()
            writes.append(
                pltpu.async_copy(
                    vals_v,
                    out3.at[d // 8, lax.rem(d, 8), pl.ds(sid * b_per_s, b_per_s)],
                    osem,
                )
            )
            writes[-1].wait()
            plsc.subcore_barrier()

    return emb_kernel


def kernel(cls, cls_emb):
    (B,) = cls.shape
    V, D = cls_emb.shape
    info = plsc.get_sparse_core_info()
    NC, NS = info.num_cores, info.num_subcores
    idx = cls.astype(jnp.int32)
    tab2 = cls_emb.T
    out3 = _make_emb_kernel(B, V, D, NC, NS)(idx, tab2)
    return out3.reshape(D, B).T


# speedup vs baseline: 5.1043x; 5.1043x over previous
import functools

import jax
import jax.numpy as jnp
from jax import lax
from jax.experimental import pallas as pl
from jax.experimental.pallas import tpu as pltpu
from jax.experimental.pallas import tpu_sc as plsc

_CHUNK = 128


def _make_emb_kernel(B, V, D, NC, NS):
    NW = NC * NS
    b_per_w = B // NW
    n_chunks = b_per_w // _CHUNK

    mesh = plsc.VectorSubcoreMesh(core_axis_name="c", subcore_axis_name="s")

    @functools.partial(
        pl.kernel,
        out_type=jax.ShapeDtypeStruct((B, D), jnp.float32),
        mesh=mesh,
        scratch_types=[
            pltpu.VMEM((n_chunks, _CHUNK), jnp.int32),
            pltpu.VMEM((b_per_w, D), jnp.float32),
            pltpu.SemaphoreType.DMA,
        ],
        compiler_params=pltpu.CompilerParams(use_tc_tiling_on_sc=False),
    )
    def emb_kernel(idx_hbm, table_hbm, out_hbm, idx_v, rows_v, sem):
        wid = lax.axis_index("s") * NC + lax.axis_index("c")
        base = wid * b_per_w
        pltpu.sync_copy(idx_hbm.at[wid], idx_v)
        gathers = []
        for j in range(n_chunks):
            gathers.append(
                pltpu.async_copy(
                    table_hbm.at[idx_v.at[j]],
                    rows_v.at[pl.ds(j * _CHUNK, _CHUNK)],
                    sem,
                )
            )
        for g in gathers:
            g.wait()
        pltpu.sync_copy(rows_v, out_hbm.at[pl.ds(base, b_per_w)])

    return emb_kernel


def kernel(cls, cls_emb):
    (B,) = cls.shape
    V, D = cls_emb.shape
    info = plsc.get_sparse_core_info()
    NC, NS = info.num_cores, info.num_subcores
    NW = NC * NS
    idx = cls.astype(jnp.int32).reshape(NW, B // (NW * _CHUNK), _CHUNK)
    return _make_emb_kernel(B, V, D, NC, NS)(idx, cls_emb)
